# baseline (speedup 1.0000x reference)
import functools

import numpy as np
import jax
import jax.numpy as jnp
from jax import lax
from jax.experimental import pallas as pl
from jax.experimental.pallas import tpu as pltpu
from jax.experimental.pallas import tpu_sc as plsc

CHUNK = 128


def _sc_gather(item_table, user_table, hist_idx, user_idx, tgt_idx):
    n_pk = hist_idx.shape[0] // 2
    n_b = user_idx.shape[0]
    e = item_table.shape[1]

    info = plsc.get_sparse_core_info()
    nc, ns = info.num_cores, info.num_subcores
    nw = nc * ns

    gsz = CHUNK // 2
    ppw = n_pk // nw
    n_chunks = ppw // gsz
    bpw = n_b // nw

    hist_idx2 = hist_idx.reshape(nw, n_chunks, CHUNK)
    user_idx2 = user_idx.reshape(nw, 1, bpw)
    tgt_idx2 = tgt_idx.reshape(nw, 1, bpw)

    mesh = plsc.VectorSubcoreMesh(core_axis_name="c", subcore_axis_name="s")

    @functools.partial(
        pl.kernel,
        mesh=mesh,
        compiler_params=pltpu.CompilerParams(use_tc_tiling_on_sc=False),
        out_type=[
            jax.ShapeDtypeStruct((n_pk, 2 * e), jnp.float32),
            jax.ShapeDtypeStruct((n_b, e), jnp.float32),
            jax.ShapeDtypeStruct((n_b, e), jnp.float32),
        ],
        scratch_types=[
            pltpu.VMEM((n_chunks, CHUNK), jnp.int32),
            pltpu.VMEM((CHUNK, e), jnp.float32),
            pltpu.VMEM((CHUNK, e), jnp.float32),
            pltpu.VMEM((bpw, e), jnp.float32),
            pltpu.VMEM((1, bpw), jnp.int32),
            pltpu.SemaphoreType.DMA,
            pltpu.SemaphoreType.DMA,
            pltpu.SemaphoreType.DMA,
            pltpu.SemaphoreType.DMA,
        ],
    )
    def gather_kernel(item_tab, user_tab, h_idx, u_idx, t_idx,
                      hist_out, user_out, tgt_out,
                      idx_v, rows_a, rows_b, rows_s, idx_small,
                      ga, gb, wa, wb):
        wid = lax.axis_index("s") * nc + lax.axis_index("c")
        pltpu.sync_copy(h_idx.at[wid], idx_v)
        pbase = wid * ppw

        def start_gather(j, buf, sem):
            pltpu.async_copy(item_tab.at[idx_v.at[j]], buf, sem)

        def wait_gather(j, buf, sem):
            pltpu.make_async_copy(item_tab.at[idx_v.at[j]], buf, sem).wait()

        def out_halves(j, buf):
            row0 = pbase + j * gsz
            return ((buf.at[pl.ds(0, gsz)],
                     hist_out.at[pl.ds(row0, gsz), pl.ds(0, e)]),
                    (buf.at[pl.ds(gsz, gsz)],
                     hist_out.at[pl.ds(row0, gsz), pl.ds(e, e)]))

        def start_writes(j, buf, sem):
            for src, dst in out_halves(j, buf):
                pltpu.async_copy(src, dst, sem)

        def wait_writes(j, buf, sem):
            for src, dst in out_halves(j, buf):
                pltpu.make_async_copy(src, dst, sem).wait()

        start_gather(0, rows_a, ga)
        start_gather(1, rows_b, gb)

        def body(i, carry):
            j0 = 2 * i
            j1 = 2 * i + 1
            wait_gather(j0, rows_a, ga)
            start_writes(j0, rows_a, wa)
            wait_gather(j1, rows_b, gb)
            start_writes(j1, rows_b, wb)

            @pl.when(i < n_chunks // 2 - 1)
            def _():
                wait_writes(j0, rows_a, wa)
                start_gather(j0 + 2, rows_a, ga)
                wait_writes(j1, rows_b, wb)
                start_gather(j1 + 2, rows_b, gb)

            return carry

        lax.fori_loop(0, n_chunks // 2, body, 0)
        wait_writes(n_chunks - 2, rows_a, wa)
        wait_writes(n_chunks - 1, rows_b, wb)

        base = wid * bpw
        pltpu.sync_copy(u_idx.at[wid], idx_small)
        pltpu.async_copy(user_tab.at[idx_small.at[0]], rows_s, ga).wait()
        pltpu.sync_copy(rows_s, user_out.at[pl.ds(base, bpw)])
        pltpu.sync_copy(t_idx.at[wid], idx_small)
        pltpu.async_copy(item_tab.at[idx_small.at[0]], rows_s, ga).wait()
        pltpu.sync_copy(rows_s, tgt_out.at[pl.ds(base, bpw)])

    return gather_kernel(item_table, user_table, hist_idx2,
                         user_idx2, tgt_idx2)


def _tc_body(bt, t_real, ts, e,
             hist_ref, te_ref, ue_ref, hl_ref, uf_ref, if_ref,
             wuf_ref, buf_ref, wif_ref, bif_ref,
             wa1_ref, ba1_ref, wa2_ref, ba2_ref, wa3_ref, ba3_ref,
             wu1_ref, bu1_ref, wu2_ref, bu2_ref, wu3_ref,
             wi1_ref, bi1_ref, wi2_ref, bi2_ref, wi3_ref,
             out_ref):
    f32 = jnp.float32
    th = ts // 2
    q = te_ref[...]
    xp = hist_ref[...]
    wa1 = wa1_ref[...]
    a_blk = wa1[0:e]
    b_blk = wa1[e:2 * e]
    c_blk = wa1[2 * e:3 * e]
    d_blk = wa1[3 * e:4 * e]

    ze = jnp.zeros((e, e), f32)
    bc = b_blk - c_blk
    w_top = jnp.concatenate([jnp.concatenate([bc, ze], 1),
                             jnp.concatenate([ze, bc], 1)], 0)
    w_bot = jnp.concatenate([jnp.concatenate([d_blk, ze], 1),
                             jnp.concatenate([ze, d_blk], 1)], 0)

    q2 = jnp.concatenate([q, q], axis=1)
    xp3 = xp.reshape(bt, th, 2 * e)
    prodp = xp3 * q2[:, None, :]

    y = (jnp.dot(xp, w_top, preferred_element_type=f32)
         + jnp.dot(prodp.reshape(bt * th, 2 * e), w_bot,
                   preferred_element_type=f32))
    qpart = jnp.dot(q, a_blk + c_blk, preferred_element_type=f32) + ba1_ref[...]
    qp2 = jnp.concatenate([qpart, qpart], axis=1)
    h1 = jax.nn.sigmoid(y.reshape(bt, th, 2 * e) + qp2[:, None, :])

    wa2 = wa2_ref[...]
    nh = wa2.shape[1]
    z16 = jnp.zeros((e, nh), f32)
    w22 = jnp.concatenate([jnp.concatenate([wa2, z16], 1),
                           jnp.concatenate([z16, wa2], 1)], 0)
    ba2 = ba2_ref[...]
    ba22 = jnp.concatenate([ba2, ba2], axis=1)
    h2 = jax.nn.sigmoid(
        jnp.dot(h1.reshape(bt * th, 2 * e), w22, preferred_element_type=f32)
        + ba22)
    h23 = h2.reshape(bt, th, 2 * nh)
    wa3 = wa3_ref[...]
    z1 = jnp.zeros((1, nh), f32)
    wa3e = jnp.concatenate([wa3, z1], 1)[None, :, :]
    wa3o = jnp.concatenate([z1, wa3], 1)[None, :, :]
    se = jnp.sum(h23 * wa3e, axis=-1)
    so = jnp.sum(h23 * wa3o, axis=-1)
    score = jnp.concatenate([se, so], axis=1) + ba3_ref[0, 0]

    hl = hl_ref[...]
    pos = lax.broadcasted_iota(jnp.int32, (bt, ts), 1)
    score = jnp.where(pos < hl, score,
                      jnp.where(pos < t_real, -1e9, -2e9))
    m = jnp.max(score, axis=1, keepdims=True)
    ex = jnp.exp(score - m)
    attn = ex / jnp.sum(ex, axis=1, keepdims=True)
    ae = attn[:, :th, None]
    ao = attn[:, th:, None]
    a2 = jnp.concatenate([jnp.broadcast_to(ae, (bt, th, e)),
                          jnp.broadcast_to(ao, (bt, th, e))], axis=-1)
    hp128 = jnp.sum(xp3 * a2, axis=1)
    history = hp128[:, :e] + hp128[:, e:]

    user_feat = jax.nn.sigmoid(
        jnp.dot(uf_ref[...], wuf_ref[...], preferred_element_type=f32) + buf_ref[...])
    item_feat = jax.nn.sigmoid(
        jnp.dot(if_ref[...], wif_ref[...], preferred_element_type=f32) + bif_ref[...])

    cu = jnp.concatenate([ue_ref[...], history, user_feat], axis=1)
    u = jax.nn.relu(jnp.dot(cu, wu1_ref[...], preferred_element_type=f32) + bu1_ref[...])
    u = jax.nn.relu(jnp.dot(u, wu2_ref[...], preferred_element_type=f32) + bu2_ref[...])
    u = jax.nn.relu(jnp.dot(u, wu3_ref[...], preferred_element_type=f32))

    ci = jnp.concatenate([q, item_feat], axis=1)
    it = jax.nn.relu(jnp.dot(ci, wi1_ref[...], preferred_element_type=f32) + bi1_ref[...])
    it = jax.nn.relu(jnp.dot(it, wi2_ref[...], preferred_element_type=f32) + bi2_ref[...])
    it = jax.nn.relu(jnp.dot(it, wi3_ref[...], preferred_element_type=f32))

    out_ref[...] = jnp.sum(u * it, axis=1, keepdims=True)


def _tc_fused(hist_pk, tgt_emb, user_emb, history_len,
              user_features, item_features, p, t_real, bt):
    b, e = tgt_emb.shape
    ts = hist_pk.shape[0] * 128 // (b * e)
    th = ts // 2
    grid = (b // bt,)

    def full(shape):
        return pl.BlockSpec(shape, lambda i: (0,) * len(shape))

    in_specs = [
        pl.BlockSpec((bt * th, 128), lambda i: (i, 0)),
        pl.BlockSpec((bt, e), lambda i: (i, 0)),
        pl.BlockSpec((bt, e), lambda i: (i, 0)),
        pl.BlockSpec((bt, 1), lambda i: (i, 0)),
        pl.BlockSpec((bt, p['W_uf'].shape[0]), lambda i: (i, 0)),
        pl.BlockSpec((bt, p['W_if'].shape[0]), lambda i: (i, 0)),
        full(p['W_uf'].shape), full((1, e)),
        full(p['W_if'].shape), full((1, e)),
        full(p['Wa1'].shape), full((1, 64)),
        full(p['Wa2'].shape), full((1, 16)),
        full((1, 16)), full((1, 1)),
        full(p['Wu1'].shape), full((1, 200)),
        full(p['Wu2'].shape), full((1, 80)),
        full(p['Wu3'].shape),
        full(p['Wi1'].shape), full((1, 200)),
        full(p['Wi2'].shape), full((1, 80)),
        full(p['Wi3'].shape),
    ]
    out_spec = pl.BlockSpec((bt, 1), lambda i: (i, 0))

    body = functools.partial(_tc_body, bt, t_real, ts, e)
    return pl.pallas_call(
        body,
        grid=grid,
        in_specs=in_specs,
        out_specs=out_spec,
        out_shape=jax.ShapeDtypeStruct((b, 1), jnp.float32),
    )(
        hist_pk, tgt_emb, user_emb, history_len.reshape(b, 1).astype(jnp.int32),
        user_features, item_features,
        p['W_uf'], p['b_uf'].reshape(1, -1),
        p['W_if'], p['b_if'].reshape(1, -1),
        p['Wa1'], p['ba1'].reshape(1, -1),
        p['Wa2'], p['ba2'].reshape(1, -1),
        p['Wa3'].reshape(1, -1), p['ba3'].reshape(1, 1),
        p['Wu1'], p['bu1'].reshape(1, -1),
        p['Wu2'], p['bu2'].reshape(1, -1),
        p['Wu3'],
        p['Wi1'], p['bi1'].reshape(1, -1),
        p['Wi2'], p['bi2'].reshape(1, -1),
        p['Wi3'],
    )


def kernel(user_id, target_item_id, history_item_id, history_len,
           user_features, item_features, params):
    p = params
    b, t = history_item_id.shape
    uid = user_id.reshape(b).astype(jnp.int32)
    tid = target_item_id.reshape(b).astype(jnp.int32)
    ts = 64
    hid = history_item_id.astype(jnp.int32)
    hid = jnp.concatenate(
        [hid, jnp.zeros((b, ts - t), jnp.int32)], axis=1)
    e_flat = hid[:, :ts // 2].reshape(-1, CHUNK // 2)
    o_flat = hid[:, ts // 2:].reshape(-1, CHUNK // 2)
    hist_idx = jnp.concatenate([e_flat, o_flat], axis=1).reshape(-1)

    hist_pk, user_emb, tgt_emb = _sc_gather(
        p['item_table'], p['user_table'], hist_idx, uid, tid)

    return _tc_fused(hist_pk, tgt_emb, user_emb, history_len,
                     user_features, item_features, p, t_real=t, bt=128)


# speedup vs baseline: 1.0008x; 1.0008x over previous
import functools

import numpy as np
import jax
import jax.numpy as jnp
from jax import lax
from jax.experimental import pallas as pl
from jax.experimental.pallas import tpu as pltpu
from jax.experimental.pallas import tpu_sc as plsc

CHUNK = 128


def _sc_gather(item_table, user_table, hist_idx, user_idx, tgt_idx):
    n_pk = hist_idx.shape[0] // 2
    n_b = user_idx.shape[0]
    e = item_table.shape[1]

    info = plsc.get_sparse_core_info()
    nc, ns = info.num_cores, info.num_subcores
    nw = nc * ns

    gsz = CHUNK // 2
    ppw = n_pk // nw
    n_chunks = ppw // gsz
    bpw = n_b // nw

    hist_idx2 = hist_idx.reshape(nw, n_chunks, CHUNK)
    user_idx2 = user_idx.reshape(nw, 1, bpw)
    tgt_idx2 = tgt_idx.reshape(nw, 1, bpw)

    mesh = plsc.VectorSubcoreMesh(core_axis_name="c", subcore_axis_name="s")

    @functools.partial(
        pl.kernel,
        mesh=mesh,
        compiler_params=pltpu.CompilerParams(use_tc_tiling_on_sc=False),
        out_type=[
            jax.ShapeDtypeStruct((n_pk, 2 * e), jnp.float32),
            jax.ShapeDtypeStruct((n_b, e), jnp.float32),
            jax.ShapeDtypeStruct((n_b, e), jnp.float32),
        ],
        scratch_types=[
            pltpu.VMEM((n_chunks, CHUNK), jnp.int32),
            pltpu.VMEM((CHUNK, e), jnp.float32),
            pltpu.VMEM((CHUNK, e), jnp.float32),
            pltpu.VMEM((bpw, e), jnp.float32),
            pltpu.VMEM((1, bpw), jnp.int32),
            pltpu.SemaphoreType.DMA,
            pltpu.SemaphoreType.DMA,
            pltpu.SemaphoreType.DMA,
            pltpu.SemaphoreType.DMA,
        ],
    )
    def gather_kernel(item_tab, user_tab, h_idx, u_idx, t_idx,
                      hist_out, user_out, tgt_out,
                      idx_v, rows_a, rows_b, rows_s, idx_small,
                      ga, gb, wa, wb):
        wid = lax.axis_index("s") * nc + lax.axis_index("c")
        pltpu.sync_copy(h_idx.at[wid], idx_v)
        pbase = wid * ppw

        def body(j, carry):
            pltpu.async_copy(item_tab.at[idx_v.at[j]], rows_a, ga).wait()
            row0 = pbase + j * gsz
            pltpu.sync_copy(rows_a.at[pl.ds(0, gsz)],
                            hist_out.at[pl.ds(row0, gsz), pl.ds(0, e)])
            pltpu.sync_copy(rows_a.at[pl.ds(gsz, gsz)],
                            hist_out.at[pl.ds(row0, gsz), pl.ds(e, e)])
            return carry

        lax.fori_loop(0, n_chunks, body, 0)

        base = wid * bpw
        pltpu.sync_copy(u_idx.at[wid], idx_small)
        pltpu.async_copy(user_tab.at[idx_small.at[0]], rows_s, ga).wait()
        pltpu.sync_copy(rows_s, user_out.at[pl.ds(base, bpw)])
        pltpu.sync_copy(t_idx.at[wid], idx_small)
        pltpu.async_copy(item_tab.at[idx_small.at[0]], rows_s, ga).wait()
        pltpu.sync_copy(rows_s, tgt_out.at[pl.ds(base, bpw)])

    return gather_kernel(item_table, user_table, hist_idx2,
                         user_idx2, tgt_idx2)


def _tc_body(bt, t_real, ts, e,
             hist_ref, te_ref, ue_ref, hl_ref, uf_ref, if_ref,
             wuf_ref, buf_ref, wif_ref, bif_ref,
             wa1_ref, ba1_ref, wa2_ref, ba2_ref, wa3_ref, ba3_ref,
             wu1_ref, bu1_ref, wu2_ref, bu2_ref, wu3_ref,
             wi1_ref, bi1_ref, wi2_ref, bi2_ref, wi3_ref,
             out_ref):
    f32 = jnp.float32
    th = ts // 2
    q = te_ref[...]
    xp = hist_ref[...]
    wa1 = wa1_ref[...]
    a_blk = wa1[0:e]
    b_blk = wa1[e:2 * e]
    c_blk = wa1[2 * e:3 * e]
    d_blk = wa1[3 * e:4 * e]

    ze = jnp.zeros((e, e), f32)
    bc = b_blk - c_blk
    w_top = jnp.concatenate([jnp.concatenate([bc, ze], 1),
                             jnp.concatenate([ze, bc], 1)], 0)
    w_bot = jnp.concatenate([jnp.concatenate([d_blk, ze], 1),
                             jnp.concatenate([ze, d_blk], 1)], 0)

    q2 = jnp.concatenate([q, q], axis=1)
    xp3 = xp.reshape(bt, th, 2 * e)
    prodp = xp3 * q2[:, None, :]

    y = (jnp.dot(xp, w_top, preferred_element_type=f32)
         + jnp.dot(prodp.reshape(bt * th, 2 * e), w_bot,
                   preferred_element_type=f32))
    qpart = jnp.dot(q, a_blk + c_blk, preferred_element_type=f32) + ba1_ref[...]
    qp2 = jnp.concatenate([qpart, qpart], axis=1)
    h1 = jax.nn.sigmoid(y.reshape(bt, th, 2 * e) + qp2[:, None, :])

    wa2 = wa2_ref[...]
    nh = wa2.shape[1]
    z16 = jnp.zeros((e, nh), f32)
    w22 = jnp.concatenate([jnp.concatenate([wa2, z16], 1),
                           jnp.concatenate([z16, wa2], 1)], 0)
    ba2 = ba2_ref[...]
    ba22 = jnp.concatenate([ba2, ba2], axis=1)
    h2 = jax.nn.sigmoid(
        jnp.dot(h1.reshape(bt * th, 2 * e), w22, preferred_element_type=f32)
        + ba22)
    h23 = h2.reshape(bt, th, 2 * nh)
    wa3 = wa3_ref[...]
    z1 = jnp.zeros((1, nh), f32)
    wa3e = jnp.concatenate([wa3, z1], 1)[None, :, :]
    wa3o = jnp.concatenate([z1, wa3], 1)[None, :, :]
    se = jnp.sum(h23 * wa3e, axis=-1)
    so = jnp.sum(h23 * wa3o, axis=-1)
    score = jnp.concatenate([se, so], axis=1) + ba3_ref[0, 0]

    hl = hl_ref[...]
    pos = lax.broadcasted_iota(jnp.int32, (bt, ts), 1)
    score = jnp.where(pos < hl, score,
                      jnp.where(pos < t_real, -1e9, -2e9))
    m = jnp.max(score, axis=1, keepdims=True)
    ex = jnp.exp(score - m)
    attn = ex / jnp.sum(ex, axis=1, keepdims=True)
    ae = attn[:, :th, None]
    ao = attn[:, th:, None]
    a2 = jnp.concatenate([jnp.broadcast_to(ae, (bt, th, e)),
                          jnp.broadcast_to(ao, (bt, th, e))], axis=-1)
    hp128 = jnp.sum(xp3 * a2, axis=1)
    history = hp128[:, :e] + hp128[:, e:]

    user_feat = jax.nn.sigmoid(
        jnp.dot(uf_ref[...], wuf_ref[...], preferred_element_type=f32) + buf_ref[...])
    item_feat = jax.nn.sigmoid(
        jnp.dot(if_ref[...], wif_ref[...], preferred_element_type=f32) + bif_ref[...])

    cu = jnp.concatenate([ue_ref[...], history, user_feat], axis=1)
    u = jax.nn.relu(jnp.dot(cu, wu1_ref[...], preferred_element_type=f32) + bu1_ref[...])
    u = jax.nn.relu(jnp.dot(u, wu2_ref[...], preferred_element_type=f32) + bu2_ref[...])
    u = jax.nn.relu(jnp.dot(u, wu3_ref[...], preferred_element_type=f32))

    ci = jnp.concatenate([q, item_feat], axis=1)
    it = jax.nn.relu(jnp.dot(ci, wi1_ref[...], preferred_element_type=f32) + bi1_ref[...])
    it = jax.nn.relu(jnp.dot(it, wi2_ref[...], preferred_element_type=f32) + bi2_ref[...])
    it = jax.nn.relu(jnp.dot(it, wi3_ref[...], preferred_element_type=f32))

    out_ref[...] = jnp.sum(u * it, axis=1, keepdims=True)


def _tc_fused(hist_pk, tgt_emb, user_emb, history_len,
              user_features, item_features, p, t_real, bt):
    b, e = tgt_emb.shape
    ts = hist_pk.shape[0] * 128 // (b * e)
    th = ts // 2
    grid = (b // bt,)

    def full(shape):
        return pl.BlockSpec(shape, lambda i: (0,) * len(shape))

    in_specs = [
        pl.BlockSpec((bt * th, 128), lambda i: (i, 0)),
        pl.BlockSpec((bt, e), lambda i: (i, 0)),
        pl.BlockSpec((bt, e), lambda i: (i, 0)),
        pl.BlockSpec((bt, 1), lambda i: (i, 0)),
        pl.BlockSpec((bt, p['W_uf'].shape[0]), lambda i: (i, 0)),
        pl.BlockSpec((bt, p['W_if'].shape[0]), lambda i: (i, 0)),
        full(p['W_uf'].shape), full((1, e)),
        full(p['W_if'].shape), full((1, e)),
        full(p['Wa1'].shape), full((1, 64)),
        full(p['Wa2'].shape), full((1, 16)),
        full((1, 16)), full((1, 1)),
        full(p['Wu1'].shape), full((1, 200)),
        full(p['Wu2'].shape), full((1, 80)),
        full(p['Wu3'].shape),
        full(p['Wi1'].shape), full((1, 200)),
        full(p['Wi2'].shape), full((1, 80)),
        full(p['Wi3'].shape),
    ]
    out_spec = pl.BlockSpec((bt, 1), lambda i: (i, 0))

    body = functools.partial(_tc_body, bt, t_real, ts, e)
    return pl.pallas_call(
        body,
        grid=grid,
        in_specs=in_specs,
        out_specs=out_spec,
        out_shape=jax.ShapeDtypeStruct((b, 1), jnp.float32),
    )(
        hist_pk, tgt_emb, user_emb, history_len.reshape(b, 1).astype(jnp.int32),
        user_features, item_features,
        p['W_uf'], p['b_uf'].reshape(1, -1),
        p['W_if'], p['b_if'].reshape(1, -1),
        p['Wa1'], p['ba1'].reshape(1, -1),
        p['Wa2'], p['ba2'].reshape(1, -1),
        p['Wa3'].reshape(1, -1), p['ba3'].reshape(1, 1),
        p['Wu1'], p['bu1'].reshape(1, -1),
        p['Wu2'], p['bu2'].reshape(1, -1),
        p['Wu3'],
        p['Wi1'], p['bi1'].reshape(1, -1),
        p['Wi2'], p['bi2'].reshape(1, -1),
        p['Wi3'],
    )


def kernel(user_id, target_item_id, history_item_id, history_len,
           user_features, item_features, params):
    p = params
    b, t = history_item_id.shape
    uid = user_id.reshape(b).astype(jnp.int32)
    tid = target_item_id.reshape(b).astype(jnp.int32)
    ts = 64
    hid = history_item_id.astype(jnp.int32)
    hid = jnp.concatenate(
        [hid, jnp.zeros((b, ts - t), jnp.int32)], axis=1)
    e_flat = hid[:, :ts // 2].reshape(-1, CHUNK // 2)
    o_flat = hid[:, ts // 2:].reshape(-1, CHUNK // 2)
    hist_idx = jnp.concatenate([e_flat, o_flat], axis=1).reshape(-1)

    hist_pk, user_emb, tgt_emb = _sc_gather(
        p['item_table'], p['user_table'], hist_idx, uid, tid)

    return _tc_fused(hist_pk, tgt_emb, user_emb, history_len,
                     user_features, item_features, p, t_real=t, bt=128)


# speedup vs baseline: 3.9133x; 3.9102x over previous
import functools

import numpy as np
import jax
import jax.numpy as jnp
from jax import lax
from jax.experimental import pallas as pl
from jax.experimental.pallas import tpu as pltpu
from jax.experimental.pallas import tpu_sc as plsc

CHUNK = 128


def _sc_gather(item_table, user_table, hist_idx, user_idx, tgt_idx):
    n_pk = hist_idx.size // 2
    n_b = user_idx.shape[0]
    e = item_table.shape[1]

    info = plsc.get_sparse_core_info()
    nc, ns = info.num_cores, info.num_subcores
    nw = nc * ns

    gsz = CHUNK // 2
    ppw = n_pk // nw
    n_chunks = ppw // gsz
    bpw = n_b // nw

    hist_idx2 = hist_idx.reshape(nw * n_chunks, CHUNK)
    user_idx2 = user_idx.reshape(nw, 1, bpw)
    tgt_idx2 = tgt_idx.reshape(nw, 1, bpw)

    mesh = plsc.VectorSubcoreMesh(core_axis_name="c", subcore_axis_name="s")

    @functools.partial(
        pl.kernel,
        mesh=mesh,
        compiler_params=pltpu.CompilerParams(use_tc_tiling_on_sc=False),
        out_type=[
            jax.ShapeDtypeStruct((n_pk, 2 * e), jnp.float32),
            jax.ShapeDtypeStruct((n_b, e), jnp.float32),
            jax.ShapeDtypeStruct((n_b, e), jnp.float32),
        ],
        scratch_types=[
            pltpu.VMEM((n_chunks, CHUNK), jnp.int32),
            pltpu.VMEM((CHUNK,), jnp.int32),
            pltpu.VMEM((CHUNK, e), jnp.float32),
            pltpu.VMEM((CHUNK, e), jnp.float32),
            pltpu.VMEM((bpw, e), jnp.float32),
            pltpu.VMEM((1, bpw), jnp.int32),
            pltpu.SemaphoreType.DMA,
            pltpu.SemaphoreType.DMA,
        ],
    )
    def gather_kernel(item_tab, user_tab, h_idx, u_idx, t_idx,
                      hist_out, user_out, tgt_out,
                      idx_v, idx_b, rows_a, rows_b, rows_s, idx_small,
                      ga, gb):
        wid = lax.axis_index("s") * nc + lax.axis_index("c")
        pltpu.sync_copy(h_idx.at[pl.ds(wid * n_chunks, n_chunks)], idx_v)
        pbase = wid * ppw

        def body(j, carry):
            for dst, src in ((0, 0), (16, 16),
                             (32, 64), (48, 80),
                             (64, 32), (80, 48),
                             (96, 96), (112, 112)):
                idx_b[pl.ds(dst, 16)] = idx_v[j, pl.ds(src, 16)]
            pltpu.async_copy(item_tab.at[idx_b], rows_a, ga).wait()
            row0 = pbase + j * gsz
            pltpu.sync_copy(rows_a.at[pl.ds(0, gsz)],
                            hist_out.at[pl.ds(row0, gsz), pl.ds(0, e)])
            pltpu.sync_copy(rows_a.at[pl.ds(gsz, gsz)],
                            hist_out.at[pl.ds(row0, gsz), pl.ds(e, e)])
            return carry

        lax.fori_loop(0, n_chunks, body, 0)

        base = wid * bpw
        pltpu.sync_copy(u_idx.at[wid], idx_small)
        pltpu.async_copy(user_tab.at[idx_small.at[0]], rows_s, ga).wait()
        pltpu.sync_copy(rows_s, user_out.at[pl.ds(base, bpw)])
        pltpu.sync_copy(t_idx.at[wid], idx_small)
        pltpu.async_copy(item_tab.at[idx_small.at[0]], rows_s, ga).wait()
        pltpu.sync_copy(rows_s, tgt_out.at[pl.ds(base, bpw)])

    return gather_kernel(item_table, user_table, hist_idx2,
                         user_idx2, tgt_idx2)


def _tc_body(bt, t_real, ts, e,
             hist_ref, te_ref, ue_ref, hl_ref, uf_ref, if_ref,
             wuf_ref, buf_ref, wif_ref, bif_ref,
             wa1_ref, ba1_ref, wa2_ref, ba2_ref, wa3_ref, ba3_ref,
             wu1_ref, bu1_ref, wu2_ref, bu2_ref, wu3_ref,
             wi1_ref, bi1_ref, wi2_ref, bi2_ref, wi3_ref,
             out_ref):
    f32 = jnp.float32
    th = ts // 2
    q = te_ref[...]
    xp = hist_ref[...]
    wa1 = wa1_ref[...]
    a_blk = wa1[0:e]
    b_blk = wa1[e:2 * e]
    c_blk = wa1[2 * e:3 * e]
    d_blk = wa1[3 * e:4 * e]

    ze = jnp.zeros((e, e), f32)
    bc = b_blk - c_blk
    w_top = jnp.concatenate([jnp.concatenate([bc, ze], 1),
                             jnp.concatenate([ze, bc], 1)], 0)
    w_bot = jnp.concatenate([jnp.concatenate([d_blk, ze], 1),
                             jnp.concatenate([ze, d_blk], 1)], 0)

    q2 = jnp.concatenate([q, q], axis=1)
    xp3 = xp.reshape(bt, th, 2 * e)
    prodp = xp3 * q2[:, None, :]

    y = (jnp.dot(xp, w_top, preferred_element_type=f32)
         + jnp.dot(prodp.reshape(bt * th, 2 * e), w_bot,
                   preferred_element_type=f32))
    qpart = jnp.dot(q, a_blk + c_blk, preferred_element_type=f32) + ba1_ref[...]
    qp2 = jnp.concatenate([qpart, qpart], axis=1)
    h1 = jax.nn.sigmoid(y.reshape(bt, th, 2 * e) + qp2[:, None, :])

    wa2 = wa2_ref[...]
    nh = wa2.shape[1]
    z16 = jnp.zeros((e, nh), f32)
    w22 = jnp.concatenate([jnp.concatenate([wa2, z16], 1),
                           jnp.concatenate([z16, wa2], 1)], 0)
    ba2 = ba2_ref[...]
    ba22 = jnp.concatenate([ba2, ba2], axis=1)
    h2 = jax.nn.sigmoid(
        jnp.dot(h1.reshape(bt * th, 2 * e), w22, preferred_element_type=f32)
        + ba22)
    h23 = h2.reshape(bt, th, 2 * nh)
    wa3 = wa3_ref[...]
    z1 = jnp.zeros((1, nh), f32)
    wa3e = jnp.concatenate([wa3, z1], 1)[None, :, :]
    wa3o = jnp.concatenate([z1, wa3], 1)[None, :, :]
    se = jnp.sum(h23 * wa3e, axis=-1)
    so = jnp.sum(h23 * wa3o, axis=-1)
    score = jnp.concatenate([se, so], axis=1) + ba3_ref[0, 0]

    hl = hl_ref[...]
    pos = lax.broadcasted_iota(jnp.int32, (bt, ts), 1)
    score = jnp.where(pos < hl, score,
                      jnp.where(pos < t_real, -1e9, -2e9))
    m = jnp.max(score, axis=1, keepdims=True)
    ex = jnp.exp(score - m)
    attn = ex / jnp.sum(ex, axis=1, keepdims=True)
    ae = attn[:, :th, None]
    ao = attn[:, th:, None]
    a2 = jnp.concatenate([jnp.broadcast_to(ae, (bt, th, e)),
                          jnp.broadcast_to(ao, (bt, th, e))], axis=-1)
    hp128 = jnp.sum(xp3 * a2, axis=1)
    history = hp128[:, :e] + hp128[:, e:]

    user_feat = jax.nn.sigmoid(
        jnp.dot(uf_ref[...], wuf_ref[...], preferred_element_type=f32) + buf_ref[...])
    item_feat = jax.nn.sigmoid(
        jnp.dot(if_ref[...], wif_ref[...], preferred_element_type=f32) + bif_ref[...])

    cu = jnp.concatenate([ue_ref[...], history, user_feat], axis=1)
    u = jax.nn.relu(jnp.dot(cu, wu1_ref[...], preferred_element_type=f32) + bu1_ref[...])
    u = jax.nn.relu(jnp.dot(u, wu2_ref[...], preferred_element_type=f32) + bu2_ref[...])
    u = jax.nn.relu(jnp.dot(u, wu3_ref[...], preferred_element_type=f32))

    ci = jnp.concatenate([q, item_feat], axis=1)
    it = jax.nn.relu(jnp.dot(ci, wi1_ref[...], preferred_element_type=f32) + bi1_ref[...])
    it = jax.nn.relu(jnp.dot(it, wi2_ref[...], preferred_element_type=f32) + bi2_ref[...])
    it = jax.nn.relu(jnp.dot(it, wi3_ref[...], preferred_element_type=f32))

    out_ref[...] = jnp.sum(u * it, axis=1, keepdims=True)


def _tc_fused(hist_pk, tgt_emb, user_emb, history_len,
              user_features, item_features, p, t_real, bt):
    b, e = tgt_emb.shape
    ts = hist_pk.shape[0] * 128 // (b * e)
    th = ts // 2
    grid = (b // bt,)

    def full(shape):
        return pl.BlockSpec(shape, lambda i: (0,) * len(shape))

    in_specs = [
        pl.BlockSpec((bt * th, 128), lambda i: (i, 0)),
        pl.BlockSpec((bt, e), lambda i: (i, 0)),
        pl.BlockSpec((bt, e), lambda i: (i, 0)),
        pl.BlockSpec((bt, 1), lambda i: (i, 0)),
        pl.BlockSpec((bt, p['W_uf'].shape[0]), lambda i: (i, 0)),
        pl.BlockSpec((bt, p['W_if'].shape[0]), lambda i: (i, 0)),
        full(p['W_uf'].shape), full((1, e)),
        full(p['W_if'].shape), full((1, e)),
        full(p['Wa1'].shape), full((1, 64)),
        full(p['Wa2'].shape), full((1, 16)),
        full((1, 16)), full((1, 1)),
        full(p['Wu1'].shape), full((1, 200)),
        full(p['Wu2'].shape), full((1, 80)),
        full(p['Wu3'].shape),
        full(p['Wi1'].shape), full((1, 200)),
        full(p['Wi2'].shape), full((1, 80)),
        full(p['Wi3'].shape),
    ]
    out_spec = pl.BlockSpec((bt, 1), lambda i: (i, 0))

    body = functools.partial(_tc_body, bt, t_real, ts, e)
    return pl.pallas_call(
        body,
        grid=grid,
        in_specs=in_specs,
        out_specs=out_spec,
        out_shape=jax.ShapeDtypeStruct((b, 1), jnp.float32),
    )(
        hist_pk, tgt_emb, user_emb, history_len.reshape(b, 1).astype(jnp.int32),
        user_features, item_features,
        p['W_uf'], p['b_uf'].reshape(1, -1),
        p['W_if'], p['b_if'].reshape(1, -1),
        p['Wa1'], p['ba1'].reshape(1, -1),
        p['Wa2'], p['ba2'].reshape(1, -1),
        p['Wa3'].reshape(1, -1), p['ba3'].reshape(1, 1),
        p['Wu1'], p['bu1'].reshape(1, -1),
        p['Wu2'], p['bu2'].reshape(1, -1),
        p['Wu3'],
        p['Wi1'], p['bi1'].reshape(1, -1),
        p['Wi2'], p['bi2'].reshape(1, -1),
        p['Wi3'],
    )


def kernel(user_id, target_item_id, history_item_id, history_len,
           user_features, item_features, params):
    p = params
    b, t = history_item_id.shape
    uid = user_id.reshape(b).astype(jnp.int32)
    tid = target_item_id.reshape(b).astype(jnp.int32)
    ts = 64
    hid = history_item_id.astype(jnp.int32)
    hid = jnp.concatenate([hid, hid[:, :ts - t]], axis=1)
    hist_idx = hid.reshape(b // 2, 2 * ts)

    hist_pk, user_emb, tgt_emb = _sc_gather(
        p['item_table'], p['user_table'], hist_idx, uid, tid)

    return _tc_fused(hist_pk, tgt_emb, user_emb, history_len,
                     user_features, item_features, p, t_real=t, bt=128)


# speedup vs baseline: 4.3912x; 1.1221x over previous
import functools

import numpy as np
import jax
import jax.numpy as jnp
from jax import lax
from jax.experimental import pallas as pl
from jax.experimental.pallas import tpu as pltpu
from jax.experimental.pallas import tpu_sc as plsc

CHUNK = 128


def _sc_gather(item_table, user_table, hist_idx, user_idx, tgt_idx):
    n_pk = hist_idx.size // 2
    n_b = user_idx.shape[0]
    e = item_table.shape[1]

    info = plsc.get_sparse_core_info()
    nc, ns = info.num_cores, info.num_subcores
    nw = nc * ns

    gsz = CHUNK // 2
    ppw = n_pk // nw
    n_chunks = ppw // gsz
    bpw = n_b // nw

    hist_idx2 = hist_idx.reshape(nw * n_chunks, CHUNK)
    user_idx2 = user_idx.reshape(nw, 1, bpw)
    tgt_idx2 = tgt_idx.reshape(nw, 1, bpw)

    mesh = plsc.VectorSubcoreMesh(core_axis_name="c", subcore_axis_name="s")

    @functools.partial(
        pl.kernel,
        mesh=mesh,
        compiler_params=pltpu.CompilerParams(use_tc_tiling_on_sc=False),
        out_type=[
            jax.ShapeDtypeStruct((n_pk, 2 * e), jnp.float32),
            jax.ShapeDtypeStruct((n_b, e), jnp.float32),
            jax.ShapeDtypeStruct((n_b, e), jnp.float32),
        ],
        scratch_types=[
            pltpu.VMEM((n_chunks, CHUNK), jnp.int32),
            pltpu.VMEM((n_chunks, CHUNK), jnp.int32),
            pltpu.VMEM((CHUNK, e), jnp.float32),
            pltpu.VMEM((CHUNK, e), jnp.float32),
            pltpu.VMEM((CHUNK, e), jnp.float32),
            pltpu.VMEM((CHUNK, e), jnp.float32),
            pltpu.VMEM((bpw, e), jnp.float32),
            pltpu.VMEM((1, bpw), jnp.int32),
            pltpu.SemaphoreType.DMA,
            pltpu.SemaphoreType.DMA,
            pltpu.SemaphoreType.DMA,
            pltpu.SemaphoreType.DMA,
            pltpu.SemaphoreType.DMA,
            pltpu.SemaphoreType.DMA,
        ],
    )
    def gather_kernel(item_tab, user_tab, h_idx, u_idx, t_idx,
                      hist_out, user_out, tgt_out,
                      idx_v, idx_b, r0, r1, r2, r3, rows_s, idx_small,
                      g0, g1, w0, w1, w2, w3):
        wid = lax.axis_index("s") * nc + lax.axis_index("c")
        pltpu.sync_copy(h_idx.at[pl.ds(wid * n_chunks, n_chunks)], idx_v)
        pbase = wid * ppw
        bufs = (r0, r1, r2, r3)
        gsems = (g0, g1)
        wsems = (w0, w1, w2, w3)
        G = 8

        def shuffle(j, carry):
            for dst, src in ((0, 0), (16, 16),
                             (32, 64), (48, 80),
                             (64, 32), (80, 48),
                             (96, 96), (112, 112)):
                idx_b[j, pl.ds(dst, 16)] = idx_v[j, pl.ds(src, 16)]
            return carry

        lax.fori_loop(0, n_chunks, shuffle, 0)

        def start_gather(j, u):
            return pltpu.async_copy(item_tab.at[idx_b.at[j]],
                                    bufs[u % 4], gsems[u % 2])

        def start_writes(j, u):
            row0 = pbase + j * gsz
            buf = bufs[u % 4]
            c0 = pltpu.async_copy(buf.at[pl.ds(0, gsz)],
                                  hist_out.at[pl.ds(row0, gsz), pl.ds(0, e)],
                                  wsems[u % 4])
            c1 = pltpu.async_copy(buf.at[pl.ds(gsz, gsz)],
                                  hist_out.at[pl.ds(row0, gsz), pl.ds(e, e)],
                                  wsems[u % 4])
            return c0, c1

        def group(g, carry):
            jb = g * G
            hs = {0: start_gather(jb, 0)}
            ws = {}
            for u in range(G):
                if u + 1 < G:
                    if u + 1 >= 4:
                        for c in ws[u - 3]:
                            c.wait()
                    hs[u + 1] = start_gather(jb + u + 1, u + 1)
                hs[u].wait()
                ws[u] = start_writes(jb + u, u)
            for u in range(G - 4, G):
                for c in ws[u]:
                    c.wait()
            return carry

        lax.fori_loop(0, n_chunks // G, group, 0)

        base = wid * bpw
        pltpu.sync_copy(u_idx.at[wid], idx_small)
        pltpu.async_copy(user_tab.at[idx_small.at[0]], rows_s, g0).wait()
        pltpu.sync_copy(rows_s, user_out.at[pl.ds(base, bpw)])
        pltpu.sync_copy(t_idx.at[wid], idx_small)
        pltpu.async_copy(item_tab.at[idx_small.at[0]], rows_s, g0).wait()
        pltpu.sync_copy(rows_s, tgt_out.at[pl.ds(base, bpw)])

    return gather_kernel(item_table, user_table, hist_idx2,
                         user_idx2, tgt_idx2)


def _tc_body(bt, t_real, ts, e,
             hist_ref, te_ref, ue_ref, hl_ref, uf_ref, if_ref,
             wuf_ref, buf_ref, wif_ref, bif_ref,
             wa1_ref, ba1_ref, wa2_ref, ba2_ref, wa3_ref, ba3_ref,
             wu1_ref, bu1_ref, wu2_ref, bu2_ref, wu3_ref,
             wi1_ref, bi1_ref, wi2_ref, bi2_ref, wi3_ref,
             out_ref):
    f32 = jnp.float32
    th = ts // 2
    q = te_ref[...]
    xp = hist_ref[...]
    wa1 = wa1_ref[...]
    a_blk = wa1[0:e]
    b_blk = wa1[e:2 * e]
    c_blk = wa1[2 * e:3 * e]
    d_blk = wa1[3 * e:4 * e]

    ze = jnp.zeros((e, e), f32)
    bc = b_blk - c_blk
    w_top = jnp.concatenate([jnp.concatenate([bc, ze], 1),
                             jnp.concatenate([ze, bc], 1)], 0)
    w_bot = jnp.concatenate([jnp.concatenate([d_blk, ze], 1),
                             jnp.concatenate([ze, d_blk], 1)], 0)

    q2 = jnp.concatenate([q, q], axis=1)
    xp3 = xp.reshape(bt, th, 2 * e)
    prodp = xp3 * q2[:, None, :]

    y = (jnp.dot(xp, w_top, preferred_element_type=f32)
         + jnp.dot(prodp.reshape(bt * th, 2 * e), w_bot,
                   preferred_element_type=f32))
    qpart = jnp.dot(q, a_blk + c_blk, preferred_element_type=f32) + ba1_ref[...]
    qp2 = jnp.concatenate([qpart, qpart], axis=1)
    h1 = jax.nn.sigmoid(y.reshape(bt, th, 2 * e) + qp2[:, None, :])

    wa2 = wa2_ref[...]
    nh = wa2.shape[1]
    z16 = jnp.zeros((e, nh), f32)
    w22 = jnp.concatenate([jnp.concatenate([wa2, z16], 1),
                           jnp.concatenate([z16, wa2], 1)], 0)
    ba2 = ba2_ref[...]
    ba22 = jnp.concatenate([ba2, ba2], axis=1)
    h2 = jax.nn.sigmoid(
        jnp.dot(h1.reshape(bt * th, 2 * e), w22, preferred_element_type=f32)
        + ba22)
    h23 = h2.reshape(bt, th, 2 * nh)
    wa3 = wa3_ref[...]
    z1 = jnp.zeros((1, nh), f32)
    wa3e = jnp.concatenate([wa3, z1], 1)[None, :, :]
    wa3o = jnp.concatenate([z1, wa3], 1)[None, :, :]
    se = jnp.sum(h23 * wa3e, axis=-1)
    so = jnp.sum(h23 * wa3o, axis=-1)
    score = jnp.concatenate([se, so], axis=1) + ba3_ref[0, 0]

    hl = hl_ref[...]
    pos = lax.broadcasted_iota(jnp.int32, (bt, ts), 1)
    score = jnp.where(pos < hl, score,
                      jnp.where(pos < t_real, -1e9, -2e9))
    m = jnp.max(score, axis=1, keepdims=True)
    ex = jnp.exp(score - m)
    attn = ex / jnp.sum(ex, axis=1, keepdims=True)
    ae = attn[:, :th, None]
    ao = attn[:, th:, None]
    a2 = jnp.concatenate([jnp.broadcast_to(ae, (bt, th, e)),
                          jnp.broadcast_to(ao, (bt, th, e))], axis=-1)
    hp128 = jnp.sum(xp3 * a2, axis=1)
    history = hp128[:, :e] + hp128[:, e:]

    user_feat = jax.nn.sigmoid(
        jnp.dot(uf_ref[...], wuf_ref[...], preferred_element_type=f32) + buf_ref[...])
    item_feat = jax.nn.sigmoid(
        jnp.dot(if_ref[...], wif_ref[...], preferred_element_type=f32) + bif_ref[...])

    cu = jnp.concatenate([ue_ref[...], history, user_feat], axis=1)
    u = jax.nn.relu(jnp.dot(cu, wu1_ref[...], preferred_element_type=f32) + bu1_ref[...])
    u = jax.nn.relu(jnp.dot(u, wu2_ref[...], preferred_element_type=f32) + bu2_ref[...])
    u = jax.nn.relu(jnp.dot(u, wu3_ref[...], preferred_element_type=f32))

    ci = jnp.concatenate([q, item_feat], axis=1)
    it = jax.nn.relu(jnp.dot(ci, wi1_ref[...], preferred_element_type=f32) + bi1_ref[...])
    it = jax.nn.relu(jnp.dot(it, wi2_ref[...], preferred_element_type=f32) + bi2_ref[...])
    it = jax.nn.relu(jnp.dot(it, wi3_ref[...], preferred_element_type=f32))

    out_ref[...] = jnp.sum(u * it, axis=1, keepdims=True)


def _tc_fused(hist_pk, tgt_emb, user_emb, history_len,
              user_features, item_features, p, t_real, bt):
    b, e = tgt_emb.shape
    ts = hist_pk.shape[0] * 128 // (b * e)
    th = ts // 2
    grid = (b // bt,)

    def full(shape):
        return pl.BlockSpec(shape, lambda i: (0,) * len(shape))

    in_specs = [
        pl.BlockSpec((bt * th, 128), lambda i: (i, 0)),
        pl.BlockSpec((bt, e), lambda i: (i, 0)),
        pl.BlockSpec((bt, e), lambda i: (i, 0)),
        pl.BlockSpec((bt, 1), lambda i: (i, 0)),
        pl.BlockSpec((bt, p['W_uf'].shape[0]), lambda i: (i, 0)),
        pl.BlockSpec((bt, p['W_if'].shape[0]), lambda i: (i, 0)),
        full(p['W_uf'].shape), full((1, e)),
        full(p['W_if'].shape), full((1, e)),
        full(p['Wa1'].shape), full((1, 64)),
        full(p['Wa2'].shape), full((1, 16)),
        full((1, 16)), full((1, 1)),
        full(p['Wu1'].shape), full((1, 200)),
        full(p['Wu2'].shape), full((1, 80)),
        full(p['Wu3'].shape),
        full(p['Wi1'].shape), full((1, 200)),
        full(p['Wi2'].shape), full((1, 80)),
        full(p['Wi3'].shape),
    ]
    out_spec = pl.BlockSpec((bt, 1), lambda i: (i, 0))

    body = functools.partial(_tc_body, bt, t_real, ts, e)
    return pl.pallas_call(
        body,
        grid=grid,
        in_specs=in_specs,
        out_specs=out_spec,
        out_shape=jax.ShapeDtypeStruct((b, 1), jnp.float32),
    )(
        hist_pk, tgt_emb, user_emb, history_len.reshape(b, 1).astype(jnp.int32),
        user_features, item_features,
        p['W_uf'], p['b_uf'].reshape(1, -1),
        p['W_if'], p['b_if'].reshape(1, -1),
        p['Wa1'], p['ba1'].reshape(1, -1),
        p['Wa2'], p['ba2'].reshape(1, -1),
        p['Wa3'].reshape(1, -1), p['ba3'].reshape(1, 1),
        p['Wu1'], p['bu1'].reshape(1, -1),
        p['Wu2'], p['bu2'].reshape(1, -1),
        p['Wu3'],
        p['Wi1'], p['bi1'].reshape(1, -1),
        p['Wi2'], p['bi2'].reshape(1, -1),
        p['Wi3'],
    )


def kernel(user_id, target_item_id, history_item_id, history_len,
           user_features, item_features, params):
    p = params
    b, t = history_item_id.shape
    uid = user_id.reshape(b).astype(jnp.int32)
    tid = target_item_id.reshape(b).astype(jnp.int32)
    ts = 64
    hid = history_item_id.astype(jnp.int32)
    hid = jnp.concatenate([hid, hid[:, :ts - t]], axis=1)
    hist_idx = hid.reshape(b // 2, 2 * ts)

    hist_pk, user_emb, tgt_emb = _sc_gather(
        p['item_table'], p['user_table'], hist_idx, uid, tid)

    return _tc_fused(hist_pk, tgt_emb, user_emb, history_len,
                     user_features, item_features, p, t_real=t, bt=128)


# speedup vs baseline: 4.3927x; 1.0003x over previous
import functools

import numpy as np
import jax
import jax.numpy as jnp
from jax import lax
from jax.experimental import pallas as pl
from jax.experimental.pallas import tpu as pltpu
from jax.experimental.pallas import tpu_sc as plsc

CHUNK = 128


def _sc_gather(item_table, user_table, hist_idx, user_idx, tgt_idx):
    n_pk = hist_idx.size // 2
    n_b = user_idx.shape[0]
    e = item_table.shape[1]

    info = plsc.get_sparse_core_info()
    nc, ns = info.num_cores, info.num_subcores
    nw = nc * ns

    gsz = CHUNK // 2
    ppw = n_pk // nw
    n_chunks = ppw // gsz
    bpw = n_b // nw

    ts = hist_idx.shape[1]
    hist_idx2 = hist_idx
    user_idx2 = user_idx.reshape(nw, 1, bpw)
    tgt_idx2 = tgt_idx.reshape(nw, 1, bpw)

    mesh = plsc.VectorSubcoreMesh(core_axis_name="c", subcore_axis_name="s")

    @functools.partial(
        pl.kernel,
        mesh=mesh,
        compiler_params=pltpu.CompilerParams(use_tc_tiling_on_sc=False),
        out_type=[
            jax.ShapeDtypeStruct((n_pk, 2 * e), jnp.float32),
            jax.ShapeDtypeStruct((n_b, e), jnp.float32),
            jax.ShapeDtypeStruct((n_b, e), jnp.float32),
        ],
        scratch_types=[
            pltpu.VMEM((2 * n_chunks, ts), jnp.int32),
            pltpu.VMEM((n_chunks, CHUNK), jnp.int32),
            pltpu.VMEM((CHUNK, e), jnp.float32),
            pltpu.VMEM((CHUNK, e), jnp.float32),
            pltpu.VMEM((CHUNK, e), jnp.float32),
            pltpu.VMEM((CHUNK, e), jnp.float32),
            pltpu.VMEM((bpw, e), jnp.float32),
            pltpu.VMEM((1, bpw), jnp.int32),
            pltpu.SemaphoreType.DMA,
            pltpu.SemaphoreType.DMA,
            pltpu.SemaphoreType.DMA,
            pltpu.SemaphoreType.DMA,
            pltpu.SemaphoreType.DMA,
            pltpu.SemaphoreType.DMA,
        ],
    )
    def gather_kernel(item_tab, user_tab, h_idx, u_idx, t_idx,
                      hist_out, user_out, tgt_out,
                      idx_v, idx_b, r0, r1, r2, r3, rows_s, idx_small,
                      g0, g1, w0, w1, w2, w3):
        wid = lax.axis_index("s") * nc + lax.axis_index("c")
        pltpu.sync_copy(h_idx.at[pl.ds(wid * 2 * n_chunks, 2 * n_chunks)],
                        idx_v)
        pbase = wid * ppw
        bufs = (r0, r1, r2, r3)
        gsems = (g0, g1)
        wsems = (w0, w1, w2, w3)
        G = 8

        def shuffle(j, carry):
            for dst, row_off, src in ((0, 0, 0), (16, 0, 16),
                                      (32, 1, 0), (48, 1, 16),
                                      (64, 0, 32), (80, 0, 48),
                                      (96, 1, 32), (112, 1, 48)):
                idx_b[j, pl.ds(dst, 16)] = idx_v[2 * j + row_off,
                                                 pl.ds(src, 16)]
            return carry

        lax.fori_loop(0, n_chunks, shuffle, 0)

        def start_gather(j, u):
            return pltpu.async_copy(item_tab.at[idx_b.at[j]],
                                    bufs[u % 4], gsems[u % 2])

        def start_writes(j, u):
            row0 = pbase + j * gsz
            buf = bufs[u % 4]
            c0 = pltpu.async_copy(buf.at[pl.ds(0, gsz)],
                                  hist_out.at[pl.ds(row0, gsz), pl.ds(0, e)],
                                  wsems[u % 4])
            c1 = pltpu.async_copy(buf.at[pl.ds(gsz, gsz)],
                                  hist_out.at[pl.ds(row0, gsz), pl.ds(e, e)],
                                  wsems[u % 4])
            return c0, c1

        def group(g, carry):
            jb = g * G
            hs = {0: start_gather(jb, 0)}
            ws = {}
            for u in range(G):
                if u + 1 < G:
                    if u + 1 >= 4:
                        for c in ws[u - 3]:
                            c.wait()
                    hs[u + 1] = start_gather(jb + u + 1, u + 1)
                hs[u].wait()
                ws[u] = start_writes(jb + u, u)
            for u in range(G - 4, G):
                for c in ws[u]:
                    c.wait()
            return carry

        lax.fori_loop(0, n_chunks // G, group, 0)

        base = wid * bpw
        pltpu.sync_copy(u_idx.at[wid], idx_small)
        pltpu.async_copy(user_tab.at[idx_small.at[0]], rows_s, g0).wait()
        pltpu.sync_copy(rows_s, user_out.at[pl.ds(base, bpw)])
        pltpu.sync_copy(t_idx.at[wid], idx_small)
        pltpu.async_copy(item_tab.at[idx_small.at[0]], rows_s, g0).wait()
        pltpu.sync_copy(rows_s, tgt_out.at[pl.ds(base, bpw)])

    return gather_kernel(item_table, user_table, hist_idx2,
                         user_idx2, tgt_idx2)


def _tc_body(bt, t_real, ts, e,
             hist_ref, te_ref, ue_ref, hl_ref, uf_ref, if_ref,
             wuf_ref, buf_ref, wif_ref, bif_ref,
             wa1_ref, ba1_ref, wa2_ref, ba2_ref, wa3_ref, ba3_ref,
             wu1_ref, bu1_ref, wu2_ref, bu2_ref, wu3_ref,
             wi1_ref, bi1_ref, wi2_ref, bi2_ref, wi3_ref,
             out_ref):
    f32 = jnp.float32
    th = ts // 2
    q = te_ref[...]
    xp = hist_ref[...]
    wa1 = wa1_ref[...]
    a_blk = wa1[0:e]
    b_blk = wa1[e:2 * e]
    c_blk = wa1[2 * e:3 * e]
    d_blk = wa1[3 * e:4 * e]

    ze = jnp.zeros((e, e), f32)
    bc = b_blk - c_blk
    w_top = jnp.concatenate([jnp.concatenate([bc, ze], 1),
                             jnp.concatenate([ze, bc], 1)], 0)
    w_bot = jnp.concatenate([jnp.concatenate([d_blk, ze], 1),
                             jnp.concatenate([ze, d_blk], 1)], 0)

    q2 = jnp.concatenate([q, q], axis=1)
    xp3 = xp.reshape(bt, th, 2 * e)
    prodp = xp3 * q2[:, None, :]

    y = (jnp.dot(xp, w_top, preferred_element_type=f32)
         + jnp.dot(prodp.reshape(bt * th, 2 * e), w_bot,
                   preferred_element_type=f32))
    qpart = jnp.dot(q, a_blk + c_blk, preferred_element_type=f32) + ba1_ref[...]
    qp2 = jnp.concatenate([qpart, qpart], axis=1)
    h1 = jax.nn.sigmoid(y.reshape(bt, th, 2 * e) + qp2[:, None, :])

    wa2 = wa2_ref[...]
    nh = wa2.shape[1]
    z16 = jnp.zeros((e, nh), f32)
    w22 = jnp.concatenate([jnp.concatenate([wa2, z16], 1),
                           jnp.concatenate([z16, wa2], 1)], 0)
    ba2 = ba2_ref[...]
    ba22 = jnp.concatenate([ba2, ba2], axis=1)
    h2 = jax.nn.sigmoid(
        jnp.dot(h1.reshape(bt * th, 2 * e), w22, preferred_element_type=f32)
        + ba22)
    h23 = h2.reshape(bt, th, 2 * nh)
    wa3 = wa3_ref[...]
    z1 = jnp.zeros((1, nh), f32)
    wa3e = jnp.concatenate([wa3, z1], 1)[None, :, :]
    wa3o = jnp.concatenate([z1, wa3], 1)[None, :, :]
    se = jnp.sum(h23 * wa3e, axis=-1)
    so = jnp.sum(h23 * wa3o, axis=-1)
    score = jnp.concatenate([se, so], axis=1) + ba3_ref[0, 0]

    hl = hl_ref[...]
    pos = lax.broadcasted_iota(jnp.int32, (bt, ts), 1)
    score = jnp.where(pos < hl, score,
                      jnp.where(pos < t_real, -1e9, -2e9))
    m = jnp.max(score, axis=1, keepdims=True)
    ex = jnp.exp(score - m)
    attn = ex / jnp.sum(ex, axis=1, keepdims=True)
    ae = attn[:, :th, None]
    ao = attn[:, th:, None]
    a2 = jnp.concatenate([jnp.broadcast_to(ae, (bt, th, e)),
                          jnp.broadcast_to(ao, (bt, th, e))], axis=-1)
    hp128 = jnp.sum(xp3 * a2, axis=1)
    history = hp128[:, :e] + hp128[:, e:]

    user_feat = jax.nn.sigmoid(
        jnp.dot(uf_ref[...], wuf_ref[...], preferred_element_type=f32) + buf_ref[...])
    item_feat = jax.nn.sigmoid(
        jnp.dot(if_ref[...], wif_ref[...], preferred_element_type=f32) + bif_ref[...])

    cu = jnp.concatenate([ue_ref[...], history, user_feat], axis=1)
    u = jax.nn.relu(jnp.dot(cu, wu1_ref[...], preferred_element_type=f32) + bu1_ref[...])
    u = jax.nn.relu(jnp.dot(u, wu2_ref[...], preferred_element_type=f32) + bu2_ref[...])
    u = jax.nn.relu(jnp.dot(u, wu3_ref[...], preferred_element_type=f32))

    ci = jnp.concatenate([q, item_feat], axis=1)
    it = jax.nn.relu(jnp.dot(ci, wi1_ref[...], preferred_element_type=f32) + bi1_ref[...])
    it = jax.nn.relu(jnp.dot(it, wi2_ref[...], preferred_element_type=f32) + bi2_ref[...])
    it = jax.nn.relu(jnp.dot(it, wi3_ref[...], preferred_element_type=f32))

    out_ref[...] = jnp.sum(u * it, axis=1, keepdims=True)


def _tc_fused(hist_pk, tgt_emb, user_emb, history_len,
              user_features, item_features, p, t_real, bt):
    b, e = tgt_emb.shape
    ts = hist_pk.shape[0] * 128 // (b * e)
    th = ts // 2
    grid = (b // bt,)

    def full(shape):
        return pl.BlockSpec(shape, lambda i: (0,) * len(shape))

    in_specs = [
        pl.BlockSpec((bt * th, 128), lambda i: (i, 0)),
        pl.BlockSpec((bt, e), lambda i: (i, 0)),
        pl.BlockSpec((bt, e), lambda i: (i, 0)),
        pl.BlockSpec((bt, 1), lambda i: (i, 0)),
        pl.BlockSpec((bt, p['W_uf'].shape[0]), lambda i: (i, 0)),
        pl.BlockSpec((bt, p['W_if'].shape[0]), lambda i: (i, 0)),
        full(p['W_uf'].shape), full((1, e)),
        full(p['W_if'].shape), full((1, e)),
        full(p['Wa1'].shape), full((1, 64)),
        full(p['Wa2'].shape), full((1, 16)),
        full((1, 16)), full((1, 1)),
        full(p['Wu1'].shape), full((1, 200)),
        full(p['Wu2'].shape), full((1, 80)),
        full(p['Wu3'].shape),
        full(p['Wi1'].shape), full((1, 200)),
        full(p['Wi2'].shape), full((1, 80)),
        full(p['Wi3'].shape),
    ]
    out_spec = pl.BlockSpec((bt, 1), lambda i: (i, 0))

    body = functools.partial(_tc_body, bt, t_real, ts, e)
    return pl.pallas_call(
        body,
        grid=grid,
        in_specs=in_specs,
        out_specs=out_spec,
        out_shape=jax.ShapeDtypeStruct((b, 1), jnp.float32),
    )(
        hist_pk, tgt_emb, user_emb, history_len.reshape(b, 1).astype(jnp.int32),
        user_features, item_features,
        p['W_uf'], p['b_uf'].reshape(1, -1),
        p['W_if'], p['b_if'].reshape(1, -1),
        p['Wa1'], p['ba1'].reshape(1, -1),
        p['Wa2'], p['ba2'].reshape(1, -1),
        p['Wa3'].reshape(1, -1), p['ba3'].reshape(1, 1),
        p['Wu1'], p['bu1'].reshape(1, -1),
        p['Wu2'], p['bu2'].reshape(1, -1),
        p['Wu3'],
        p['Wi1'], p['bi1'].reshape(1, -1),
        p['Wi2'], p['bi2'].reshape(1, -1),
        p['Wi3'],
    )


def kernel(user_id, target_item_id, history_item_id, history_len,
           user_features, item_features, params):
    p = params
    b, t = history_item_id.shape
    uid = user_id.reshape(b).astype(jnp.int32)
    tid = target_item_id.reshape(b).astype(jnp.int32)
    ts = 64
    hid = history_item_id.astype(jnp.int32)
    hist_idx = jnp.concatenate([hid, hid[:, :ts - t]], axis=1)

    hist_pk, user_emb, tgt_emb = _sc_gather(
        p['item_table'], p['user_table'], hist_idx, uid, tid)

    return _tc_fused(hist_pk, tgt_emb, user_emb, history_len,
                     user_features, item_features, p, t_real=t, bt=128)


# speedup vs baseline: 4.6421x; 1.0568x over previous
import functools

import numpy as np
import jax
import jax.numpy as jnp
from jax import lax
from jax.experimental import pallas as pl
from jax.experimental.pallas import tpu as pltpu
from jax.experimental.pallas import tpu_sc as plsc

CHUNK = 128


def _sc_gather(item_table, user_table, hist_idx, user_idx, tgt_idx):
    n_pk = hist_idx.size // 2
    n_b = user_idx.shape[0]
    e = item_table.shape[1]

    info = plsc.get_sparse_core_info()
    nc, ns = info.num_cores, info.num_subcores
    nw = nc * ns

    gsz = CHUNK // 2
    ppw = n_pk // nw
    n_chunks = ppw // gsz
    bpw = n_b // nw

    ts = hist_idx.shape[1]
    hist_idx2 = hist_idx
    user_idx2 = user_idx.reshape(nw, 1, bpw)
    tgt_idx2 = tgt_idx.reshape(nw, 1, bpw)

    mesh = plsc.VectorSubcoreMesh(core_axis_name="c", subcore_axis_name="s")

    @functools.partial(
        pl.kernel,
        mesh=mesh,
        compiler_params=pltpu.CompilerParams(use_tc_tiling_on_sc=False),
        out_type=[
            jax.ShapeDtypeStruct((n_pk, 2 * e), jnp.float32),
            jax.ShapeDtypeStruct((n_b, e), jnp.float32),
            jax.ShapeDtypeStruct((n_b, e), jnp.float32),
        ],
        scratch_types=[
            pltpu.VMEM((2 * n_chunks, ts), jnp.int32),
            pltpu.VMEM((n_chunks, CHUNK), jnp.int32),
            pltpu.VMEM((CHUNK, e), jnp.float32),
            pltpu.VMEM((CHUNK, e), jnp.float32),
            pltpu.VMEM((CHUNK, e), jnp.float32),
            pltpu.VMEM((CHUNK, e), jnp.float32),
            pltpu.VMEM((bpw, e), jnp.float32),
            pltpu.VMEM((1, bpw), jnp.int32),
            pltpu.SemaphoreType.DMA,
            pltpu.SemaphoreType.DMA,
            pltpu.SemaphoreType.DMA,
            pltpu.SemaphoreType.DMA,
            pltpu.SemaphoreType.DMA,
            pltpu.SemaphoreType.DMA,
        ],
    )
    def gather_kernel(item_tab, user_tab, h_idx, u_idx, t_idx,
                      hist_out, user_out, tgt_out,
                      idx_v, idx_b, r0, r1, r2, r3, rows_s, idx_small,
                      g0, g1, w0, w1, w2, w3):
        wid = lax.axis_index("s") * nc + lax.axis_index("c")
        pltpu.sync_copy(h_idx.at[pl.ds(wid * 2 * n_chunks, 2 * n_chunks)],
                        idx_v)
        pbase = wid * ppw
        bufs = (r0, r1, r2, r3)
        gsems = (g0, g1)
        wsems = (w0, w1, w2, w3)
        G = 8

        def shuffle(j, carry):
            for dst, row_off, src in ((0, 0, 0), (16, 0, 16),
                                      (32, 1, 0), (48, 1, 16),
                                      (64, 0, 32), (80, 0, 48),
                                      (96, 1, 32), (112, 1, 48)):
                idx_b[j, pl.ds(dst, 16)] = idx_v[2 * j + row_off,
                                                 pl.ds(src, 16)]
            return carry

        lax.fori_loop(0, n_chunks, shuffle, 0)

        def start_gather(j, u):
            return pltpu.async_copy(item_tab.at[idx_b.at[j]],
                                    bufs[u % 4], gsems[u % 2])

        def start_writes(j, u):
            row0 = pbase + j * gsz
            buf = bufs[u % 4]
            c0 = pltpu.async_copy(buf.at[pl.ds(0, gsz)],
                                  hist_out.at[pl.ds(row0, gsz), pl.ds(0, e)],
                                  wsems[u % 4])
            c1 = pltpu.async_copy(buf.at[pl.ds(gsz, gsz)],
                                  hist_out.at[pl.ds(row0, gsz), pl.ds(e, e)],
                                  wsems[u % 4])
            return c0, c1

        def group(g, carry):
            jb = g * G
            hs = {0: start_gather(jb, 0)}
            ws = {}
            for u in range(G):
                if u + 1 < G:
                    if u + 1 >= 4:
                        for c in ws[u - 3]:
                            c.wait()
                    hs[u + 1] = start_gather(jb + u + 1, u + 1)
                hs[u].wait()
                ws[u] = start_writes(jb + u, u)
            for u in range(G - 4, G):
                for c in ws[u]:
                    c.wait()
            return carry

        lax.fori_loop(0, n_chunks // G, group, 0)

        base = wid * bpw
        pltpu.sync_copy(u_idx.at[wid], idx_small)
        pltpu.async_copy(user_tab.at[idx_small.at[0]], rows_s, g0).wait()
        pltpu.sync_copy(rows_s, user_out.at[pl.ds(base, bpw)])
        pltpu.sync_copy(t_idx.at[wid], idx_small)
        pltpu.async_copy(item_tab.at[idx_small.at[0]], rows_s, g0).wait()
        pltpu.sync_copy(rows_s, tgt_out.at[pl.ds(base, bpw)])

    return gather_kernel(item_table, user_table, hist_idx2,
                         user_idx2, tgt_idx2)


def _tc_body(bt, t_real, ts, e,
             hist_ref, te_ref, ue_ref, hl_ref, uf_ref, if_ref,
             wuf_ref, buf_ref, wif_ref, bif_ref,
             wa1_ref, ba1_ref, wa2_ref, ba2_ref, wa3_ref, ba3_ref,
             wu1_ref, bu1_ref, wu2_ref, bu2_ref, wu3_ref,
             wi1_ref, bi1_ref, wi2_ref, bi2_ref, wi3_ref,
             out_ref):
    f32 = jnp.float32
    th = ts // 2
    q = te_ref[...]
    xp = hist_ref[...]
    wa1 = wa1_ref[...]
    a_blk = wa1[0:e]
    b_blk = wa1[e:2 * e]
    c_blk = wa1[2 * e:3 * e]
    d_blk = wa1[3 * e:4 * e]

    ze = jnp.zeros((e, e), f32)
    bc = b_blk - c_blk
    w_top = jnp.concatenate([jnp.concatenate([bc, ze], 1),
                             jnp.concatenate([ze, bc], 1)], 0)
    w_bot = jnp.concatenate([jnp.concatenate([d_blk, ze], 1),
                             jnp.concatenate([ze, d_blk], 1)], 0)

    q2 = jnp.concatenate([q, q], axis=1)
    xp3 = xp.reshape(bt, th, 2 * e)
    prodp = xp3 * q2[:, None, :]

    y = (jnp.dot(xp, w_top, preferred_element_type=f32)
         + jnp.dot(prodp.reshape(bt * th, 2 * e), w_bot,
                   preferred_element_type=f32))
    qpart = jnp.dot(q, a_blk + c_blk, preferred_element_type=f32) + ba1_ref[...]
    qp2 = jnp.concatenate([qpart, qpart], axis=1)
    h1 = jax.nn.sigmoid(y.reshape(bt, th, 2 * e) + qp2[:, None, :])

    wa2 = wa2_ref[...]
    nh = wa2.shape[1]
    z16 = jnp.zeros((e, nh), f32)
    w22 = jnp.concatenate([jnp.concatenate([wa2, z16], 1),
                           jnp.concatenate([z16, wa2], 1)], 0)
    ba2 = ba2_ref[...]
    ba22 = jnp.concatenate([ba2, ba2], axis=1)
    h2 = jax.nn.sigmoid(
        jnp.dot(h1.reshape(bt * th, 2 * e), w22, preferred_element_type=f32)
        + ba22)
    h23 = h2.reshape(bt, th, 2 * nh)
    wa3 = wa3_ref[...]
    z1 = jnp.zeros((1, nh), f32)
    wa3e = jnp.concatenate([wa3, z1], 1)[None, :, :]
    wa3o = jnp.concatenate([z1, wa3], 1)[None, :, :]
    se = jnp.sum(h23 * wa3e, axis=-1)
    so = jnp.sum(h23 * wa3o, axis=-1)
    score = jnp.concatenate([se, so], axis=1) + ba3_ref[0, 0]

    hl = hl_ref[...]
    pos = lax.broadcasted_iota(jnp.int32, (bt, ts), 1)
    score = jnp.where(pos < hl, score,
                      jnp.where(pos < t_real, -1e9, -2e9))
    m = jnp.max(score, axis=1, keepdims=True)
    ex = jnp.exp(score - m)
    attn = ex / jnp.sum(ex, axis=1, keepdims=True)
    ae = attn[:, :th, None]
    ao = attn[:, th:, None]
    a2 = jnp.concatenate([jnp.broadcast_to(ae, (bt, th, e)),
                          jnp.broadcast_to(ao, (bt, th, e))], axis=-1)
    hp128 = jnp.sum(xp3 * a2, axis=1)
    history = hp128[:, :e] + hp128[:, e:]

    user_feat = jax.nn.sigmoid(
        jnp.dot(uf_ref[...], wuf_ref[...], preferred_element_type=f32) + buf_ref[...])
    item_feat = jax.nn.sigmoid(
        jnp.dot(if_ref[...], wif_ref[...], preferred_element_type=f32) + bif_ref[...])

    cu = jnp.concatenate([ue_ref[...], history, user_feat], axis=1)
    u = jax.nn.relu(jnp.dot(cu, wu1_ref[...], preferred_element_type=f32) + bu1_ref[...])
    u = jax.nn.relu(jnp.dot(u, wu2_ref[...], preferred_element_type=f32) + bu2_ref[...])
    u = jax.nn.relu(jnp.dot(u, wu3_ref[...], preferred_element_type=f32))

    ci = jnp.concatenate([q, item_feat], axis=1)
    it = jax.nn.relu(jnp.dot(ci, wi1_ref[...], preferred_element_type=f32) + bi1_ref[...])
    it = jax.nn.relu(jnp.dot(it, wi2_ref[...], preferred_element_type=f32) + bi2_ref[...])
    it = jax.nn.relu(jnp.dot(it, wi3_ref[...], preferred_element_type=f32))

    out_ref[...] = jnp.sum(u * it, axis=1, keepdims=True)


def _tc_fused(hist_pk, tgt_emb, user_emb, history_len,
              user_features, item_features, p, t_real, bt):
    b, e = tgt_emb.shape
    ts = hist_pk.shape[0] * 128 // (b * e)
    th = ts // 2
    grid = (b // bt,)

    def full(shape):
        return pl.BlockSpec(shape, lambda i: (0,) * len(shape))

    in_specs = [
        pl.BlockSpec((bt * th, 128), lambda i: (i, 0)),
        pl.BlockSpec((bt, e), lambda i: (i, 0)),
        pl.BlockSpec((bt, e), lambda i: (i, 0)),
        pl.BlockSpec((bt, 1), lambda i: (i, 0)),
        pl.BlockSpec((bt, p['W_uf'].shape[0]), lambda i: (i, 0)),
        pl.BlockSpec((bt, p['W_if'].shape[0]), lambda i: (i, 0)),
        full(p['W_uf'].shape), full((1, e)),
        full(p['W_if'].shape), full((1, e)),
        full(p['Wa1'].shape), full((1, 64)),
        full(p['Wa2'].shape), full((1, 16)),
        full((1, 16)), full((1, 1)),
        full(p['Wu1'].shape), full((1, 200)),
        full(p['Wu2'].shape), full((1, 80)),
        full(p['Wu3'].shape),
        full(p['Wi1'].shape), full((1, 200)),
        full(p['Wi2'].shape), full((1, 80)),
        full(p['Wi3'].shape),
    ]
    out_spec = pl.BlockSpec((bt, 1), lambda i: (i, 0))

    body = functools.partial(_tc_body, bt, t_real, ts, e)
    return pl.pallas_call(
        body,
        grid=grid,
        in_specs=in_specs,
        out_specs=out_spec,
        out_shape=jax.ShapeDtypeStruct((b, 1), jnp.float32),
    )(
        hist_pk, tgt_emb, user_emb, history_len.reshape(b, 1).astype(jnp.int32),
        user_features, item_features,
        p['W_uf'], p['b_uf'].reshape(1, -1),
        p['W_if'], p['b_if'].reshape(1, -1),
        p['Wa1'], p['ba1'].reshape(1, -1),
        p['Wa2'], p['ba2'].reshape(1, -1),
        p['Wa3'].reshape(1, -1), p['ba3'].reshape(1, 1),
        p['Wu1'], p['bu1'].reshape(1, -1),
        p['Wu2'], p['bu2'].reshape(1, -1),
        p['Wu3'],
        p['Wi1'], p['bi1'].reshape(1, -1),
        p['Wi2'], p['bi2'].reshape(1, -1),
        p['Wi3'],
    )


def kernel(user_id, target_item_id, history_item_id, history_len,
           user_features, item_features, params):
    p = params
    b, t = history_item_id.shape
    uid = user_id.reshape(b).astype(jnp.int32)
    tid = target_item_id.reshape(b).astype(jnp.int32)
    ts = 64
    hid = history_item_id.astype(jnp.int32)
    hist_idx = jnp.concatenate([hid, hid[:, :ts - t]], axis=1)

    hist_pk, user_emb, tgt_emb = _sc_gather(
        p['item_table'], p['user_table'], hist_idx, uid, tid)

    return _tc_fused(hist_pk, tgt_emb, user_emb, history_len,
                     user_features, item_features, p, t_real=t, bt=256)


# speedup vs baseline: 4.7826x; 1.0303x over previous
import functools

import numpy as np
import jax
import jax.numpy as jnp
from jax import lax
from jax.experimental import pallas as pl
from jax.experimental.pallas import tpu as pltpu
from jax.experimental.pallas import tpu_sc as plsc

CHUNK = 128


def _sc_gather(item_table, user_table, hist_idx, user_idx, tgt_idx):
    n_pk = hist_idx.size // 2
    n_b = user_idx.shape[0]
    e = item_table.shape[1]

    info = plsc.get_sparse_core_info()
    nc, ns = info.num_cores, info.num_subcores
    nw = nc * ns

    gsz = CHUNK // 2
    ppw = n_pk // nw
    n_chunks = ppw // gsz
    bpw = n_b // nw

    ts = hist_idx.shape[1]
    hist_idx2 = hist_idx
    user_idx2 = user_idx.reshape(nw, 1, bpw)
    tgt_idx2 = tgt_idx.reshape(nw, 1, bpw)

    mesh = plsc.VectorSubcoreMesh(core_axis_name="c", subcore_axis_name="s")

    @functools.partial(
        pl.kernel,
        mesh=mesh,
        compiler_params=pltpu.CompilerParams(use_tc_tiling_on_sc=False),
        out_type=[
            jax.ShapeDtypeStruct((n_pk, 2 * e), jnp.float32),
            jax.ShapeDtypeStruct((n_b, e), jnp.float32),
            jax.ShapeDtypeStruct((n_b, e), jnp.float32),
        ],
        scratch_types=[
            pltpu.VMEM((2 * n_chunks, ts), jnp.int32),
            pltpu.VMEM((n_chunks, CHUNK), jnp.int32),
            pltpu.VMEM((CHUNK, e), jnp.float32),
            pltpu.VMEM((CHUNK, e), jnp.float32),
            pltpu.VMEM((CHUNK, e), jnp.float32),
            pltpu.VMEM((CHUNK, e), jnp.float32),
            pltpu.VMEM((bpw, e), jnp.float32),
            pltpu.VMEM((1, bpw), jnp.int32),
            pltpu.SemaphoreType.DMA,
            pltpu.SemaphoreType.DMA,
            pltpu.SemaphoreType.DMA,
            pltpu.SemaphoreType.DMA,
            pltpu.SemaphoreType.DMA,
            pltpu.SemaphoreType.DMA,
        ],
    )
    def gather_kernel(item_tab, user_tab, h_idx, u_idx, t_idx,
                      hist_out, user_out, tgt_out,
                      idx_v, idx_b, r0, r1, r2, r3, rows_s, idx_small,
                      g0, g1, w0, w1, w2, w3):
        wid = lax.axis_index("s") * nc + lax.axis_index("c")
        pltpu.sync_copy(h_idx.at[pl.ds(wid * 2 * n_chunks, 2 * n_chunks)],
                        idx_v)
        pbase = wid * ppw
        bufs = (r0, r1, r2, r3)
        gsems = (g0, g1)
        wsems = (w0, w1, w2, w3)
        G = 8

        def shuffle(j, carry):
            for dst, row_off, src in ((0, 0, 0), (16, 0, 16),
                                      (32, 1, 0), (48, 1, 16),
                                      (64, 0, 32), (80, 0, 48),
                                      (96, 1, 32), (112, 1, 48)):
                idx_b[j, pl.ds(dst, 16)] = idx_v[2 * j + row_off,
                                                 pl.ds(src, 16)]
            return carry

        lax.fori_loop(0, n_chunks, shuffle, 0)

        def start_gather(j, u):
            return pltpu.async_copy(item_tab.at[idx_b.at[j]],
                                    bufs[u % 4], gsems[u % 2])

        def start_writes(j, u):
            row0 = pbase + j * gsz
            buf = bufs[u % 4]
            c0 = pltpu.async_copy(buf.at[pl.ds(0, gsz)],
                                  hist_out.at[pl.ds(row0, gsz), pl.ds(0, e)],
                                  wsems[u % 4])
            c1 = pltpu.async_copy(buf.at[pl.ds(gsz, gsz)],
                                  hist_out.at[pl.ds(row0, gsz), pl.ds(e, e)],
                                  wsems[u % 4])
            return c0, c1

        def group(g, carry):
            jb = g * G
            hs = {0: start_gather(jb, 0)}
            ws = {}
            for u in range(G):
                if u + 1 < G:
                    if u + 1 >= 4:
                        for c in ws[u - 3]:
                            c.wait()
                    hs[u + 1] = start_gather(jb + u + 1, u + 1)
                hs[u].wait()
                ws[u] = start_writes(jb + u, u)
            for u in range(G - 4, G):
                for c in ws[u]:
                    c.wait()
            return carry

        lax.fori_loop(0, n_chunks // G, group, 0)

        base = wid * bpw
        pltpu.sync_copy(u_idx.at[wid], idx_small)
        pltpu.async_copy(user_tab.at[idx_small.at[0]], rows_s, g0).wait()
        pltpu.sync_copy(rows_s, user_out.at[pl.ds(base, bpw)])
        pltpu.sync_copy(t_idx.at[wid], idx_small)
        pltpu.async_copy(item_tab.at[idx_small.at[0]], rows_s, g0).wait()
        pltpu.sync_copy(rows_s, tgt_out.at[pl.ds(base, bpw)])

    return gather_kernel(item_table, user_table, hist_idx2,
                         user_idx2, tgt_idx2)


def _tc_body(bt, t_real, ts, e,
             hist_ref, te_ref, ue_ref, hl_ref, uf_ref, if_ref,
             wuf_ref, buf_ref, wif_ref, bif_ref,
             wa1_ref, ba1_ref, wa2_ref, ba2_ref, wa3_ref, ba3_ref,
             wu1_ref, bu1_ref, wu2_ref, bu2_ref, wu3_ref,
             wi1_ref, bi1_ref, wi2_ref, bi2_ref, wi3_ref,
             out_ref):
    f32 = jnp.float32
    th = ts // 2
    q = te_ref[...]
    xp = hist_ref[...]
    wa1 = wa1_ref[...]
    a_blk = wa1[0:e]
    b_blk = wa1[e:2 * e]
    c_blk = wa1[2 * e:3 * e]
    d_blk = wa1[3 * e:4 * e]

    ze = jnp.zeros((e, e), f32)
    bc = b_blk - c_blk
    w_top = jnp.concatenate([jnp.concatenate([bc, ze], 1),
                             jnp.concatenate([ze, bc], 1)], 0)
    w_bot = jnp.concatenate([jnp.concatenate([d_blk, ze], 1),
                             jnp.concatenate([ze, d_blk], 1)], 0)

    q2 = jnp.concatenate([q, q], axis=1)
    xp3 = xp.reshape(bt, th, 2 * e)
    prodp = xp3 * q2[:, None, :]

    y = (jnp.dot(xp, w_top, preferred_element_type=f32)
         + jnp.dot(prodp.reshape(bt * th, 2 * e), w_bot,
                   preferred_element_type=f32))
    qpart = jnp.dot(q, a_blk + c_blk, preferred_element_type=f32) + ba1_ref[...]
    qp2 = jnp.concatenate([qpart, qpart], axis=1)
    h1 = jax.nn.sigmoid(y.reshape(bt, th, 2 * e) + qp2[:, None, :])

    wa2 = wa2_ref[...]
    nh = wa2.shape[1]
    z16 = jnp.zeros((e, nh), f32)
    w22 = jnp.concatenate([jnp.concatenate([wa2, z16], 1),
                           jnp.concatenate([z16, wa2], 1)], 0)
    ba2 = ba2_ref[...]
    ba22 = jnp.concatenate([ba2, ba2], axis=1)
    h2 = jax.nn.sigmoid(
        jnp.dot(h1.reshape(bt * th, 2 * e), w22, preferred_element_type=f32)
        + ba22)
    h23 = h2.reshape(bt, th, 2 * nh)
    wa3 = wa3_ref[...]
    z1 = jnp.zeros((1, nh), f32)
    wa3e = jnp.concatenate([wa3, z1], 1)[None, :, :]
    wa3o = jnp.concatenate([z1, wa3], 1)[None, :, :]
    se = jnp.sum(h23 * wa3e, axis=-1)
    so = jnp.sum(h23 * wa3o, axis=-1)
    score = jnp.concatenate([se, so], axis=1) + ba3_ref[0, 0]

    hl = hl_ref[...]
    pos = lax.broadcasted_iota(jnp.int32, (bt, ts), 1)
    score = jnp.where(pos < hl, score,
                      jnp.where(pos < t_real, -1e9, -2e9))
    m = jnp.max(score, axis=1, keepdims=True)
    ex = jnp.exp(score - m)
    attn = ex / jnp.sum(ex, axis=1, keepdims=True)
    ae = attn[:, :th, None]
    ao = attn[:, th:, None]
    a2 = jnp.concatenate([jnp.broadcast_to(ae, (bt, th, e)),
                          jnp.broadcast_to(ao, (bt, th, e))], axis=-1)
    hp128 = jnp.sum(xp3 * a2, axis=1)
    history = hp128[:, :e] + hp128[:, e:]

    user_feat = jax.nn.sigmoid(
        jnp.dot(uf_ref[...], wuf_ref[...], preferred_element_type=f32) + buf_ref[...])
    item_feat = jax.nn.sigmoid(
        jnp.dot(if_ref[...], wif_ref[...], preferred_element_type=f32) + bif_ref[...])

    cu = jnp.concatenate([ue_ref[...], history, user_feat], axis=1)
    u = jax.nn.relu(jnp.dot(cu, wu1_ref[...], preferred_element_type=f32) + bu1_ref[...])
    u = jax.nn.relu(jnp.dot(u, wu2_ref[...], preferred_element_type=f32) + bu2_ref[...])
    u = jax.nn.relu(jnp.dot(u, wu3_ref[...], preferred_element_type=f32))

    ci = jnp.concatenate([q, item_feat], axis=1)
    it = jax.nn.relu(jnp.dot(ci, wi1_ref[...], preferred_element_type=f32) + bi1_ref[...])
    it = jax.nn.relu(jnp.dot(it, wi2_ref[...], preferred_element_type=f32) + bi2_ref[...])
    it = jax.nn.relu(jnp.dot(it, wi3_ref[...], preferred_element_type=f32))

    out_ref[...] = jnp.sum(u * it, axis=1, keepdims=True)


def _tc_fused(hist_pk, tgt_emb, user_emb, history_len,
              user_features, item_features, p, t_real, bt):
    b, e = tgt_emb.shape
    ts = hist_pk.shape[0] * 128 // (b * e)
    th = ts // 2
    grid = (b // bt,)

    def full(shape):
        return pl.BlockSpec(shape, lambda i: (0,) * len(shape))

    in_specs = [
        pl.BlockSpec((bt * th, 128), lambda i: (i, 0)),
        pl.BlockSpec((bt, e), lambda i: (i, 0)),
        pl.BlockSpec((bt, e), lambda i: (i, 0)),
        pl.BlockSpec((bt, 1), lambda i: (i, 0)),
        pl.BlockSpec((bt, p['W_uf'].shape[0]), lambda i: (i, 0)),
        pl.BlockSpec((bt, p['W_if'].shape[0]), lambda i: (i, 0)),
        full(p['W_uf'].shape), full((1, e)),
        full(p['W_if'].shape), full((1, e)),
        full(p['Wa1'].shape), full((1, 64)),
        full(p['Wa2'].shape), full((1, 16)),
        full((1, 16)), full((1, 1)),
        full(p['Wu1'].shape), full((1, 200)),
        full(p['Wu2'].shape), full((1, 80)),
        full(p['Wu3'].shape),
        full(p['Wi1'].shape), full((1, 200)),
        full(p['Wi2'].shape), full((1, 80)),
        full(p['Wi3'].shape),
    ]
    out_spec = pl.BlockSpec((bt, 1), lambda i: (i, 0))

    body = functools.partial(_tc_body, bt, t_real, ts, e)
    return pl.pallas_call(
        body,
        grid=grid,
        in_specs=in_specs,
        out_specs=out_spec,
        out_shape=jax.ShapeDtypeStruct((b, 1), jnp.float32),
    )(
        hist_pk, tgt_emb, user_emb, history_len.reshape(b, 1).astype(jnp.int32),
        user_features, item_features,
        p['W_uf'], p['b_uf'].reshape(1, -1),
        p['W_if'], p['b_if'].reshape(1, -1),
        p['Wa1'], p['ba1'].reshape(1, -1),
        p['Wa2'], p['ba2'].reshape(1, -1),
        p['Wa3'].reshape(1, -1), p['ba3'].reshape(1, 1),
        p['Wu1'], p['bu1'].reshape(1, -1),
        p['Wu2'], p['bu2'].reshape(1, -1),
        p['Wu3'],
        p['Wi1'], p['bi1'].reshape(1, -1),
        p['Wi2'], p['bi2'].reshape(1, -1),
        p['Wi3'],
    )


def kernel(user_id, target_item_id, history_item_id, history_len,
           user_features, item_features, params):
    p = params
    b, t = history_item_id.shape
    uid = user_id.reshape(b).astype(jnp.int32)
    tid = target_item_id.reshape(b).astype(jnp.int32)
    ts = 64
    hid = history_item_id.astype(jnp.int32)
    hist_idx = jnp.concatenate([hid, hid[:, :ts - t]], axis=1)

    hist_pk, user_emb, tgt_emb = _sc_gather(
        p['item_table'], p['user_table'], hist_idx, uid, tid)

    return _tc_fused(hist_pk, tgt_emb, user_emb, history_len,
                     user_features, item_features, p, t_real=t, bt=512)
